# baseline (speedup 1.0000x reference)
import functools

import jax
import jax.numpy as jnp
from jax import lax
from jax.experimental import pallas as pl
from jax.experimental.pallas import tpu as pltpu
from jax.experimental.pallas import tpu_sc as plsc

VOCAB = 100000
EMBED = 32
EMBED_PAD = 128
BATCH = 1024


_info = plsc.get_sparse_core_info()
_NC, _NS = _info.num_cores, _info.num_subcores
_NW = _NC * _NS
_B_PER_W = BATCH // _NW


def _make_sc_gather():
  mesh = plsc.VectorSubcoreMesh(core_axis_name="c", subcore_axis_name="s")

  @functools.partial(
      pl.kernel,
      mesh=mesh,
      compiler_params=pltpu.CompilerParams(use_tc_tiling_on_sc=True),
      out_type=jax.ShapeDtypeStruct((BATCH, EMBED_PAD), jnp.float32),
      scratch_types=[
          pltpu.VMEM((_B_PER_W,), jnp.int32),
          pltpu.VMEM((_B_PER_W, EMBED_PAD), jnp.float32),
          pltpu.SemaphoreType.DMA,
      ],
  )
  def gather_kernel(table_hbm, idx_hbm, out_hbm, idx_v, rows_v, sem):
    wid = lax.axis_index("s") * _NC + lax.axis_index("c")
    base = wid * _B_PER_W
    pltpu.sync_copy(idx_hbm.at[pl.ds(base, _B_PER_W)], idx_v)
    pltpu.async_copy(table_hbm.at[idx_v], rows_v, sem).wait()
    pltpu.sync_copy(rows_v, out_hbm.at[pl.ds(base, _B_PER_W)])

  return gather_kernel


_sc_gather = _make_sc_gather()


_VT = 2048


def _proj_body(h4_ref, w_ref, b_ref, out_ref, h_scr):
  @pl.when(pl.program_id(0) == 0)
  def _():
    h_scr[...] = h4_ref[:, :EMBED]

  out_ref[...] = (
      jnp.dot(h_scr[...], w_ref[...], preferred_element_type=jnp.float32)
      + b_ref[...]
  )


def _projection(h4, W, b):
  grid = (pl.cdiv(VOCAB, _VT),)
  return pl.pallas_call(
      _proj_body,
      grid=grid,
      in_specs=[
          pl.BlockSpec((BATCH, EMBED_PAD), lambda j: (0, 0)),
          pl.BlockSpec((EMBED, _VT), lambda j: (0, j)),
          pl.BlockSpec((_VT,), lambda j: (j,)),
      ],
      out_specs=pl.BlockSpec((BATCH, _VT), lambda j: (0, j)),
      out_shape=jax.ShapeDtypeStruct((BATCH, VOCAB), jnp.float32),
      scratch_shapes=[pltpu.VMEM((BATCH, EMBED), jnp.float32)],
  )(h4, W, b)


def kernel(x, emb_table, W, b):
  emb_pad = jnp.pad(emb_table, ((0, 0), (0, EMBED_PAD - EMBED)))
  h4 = _sc_gather(emb_pad, x.astype(jnp.int32))
  return _projection(h4, W, b)


# speedup vs baseline: 2.7994x; 2.7994x over previous
import functools

import jax
import jax.numpy as jnp
from jax import lax
from jax.experimental import pallas as pl
from jax.experimental.pallas import tpu as pltpu
from jax.experimental.pallas import tpu_sc as plsc

VOCAB = 100000
EMBED = 32
EMBED_PAD = 128
BATCH = 1024


_info = plsc.get_sparse_core_info()
_NC, _NS = _info.num_cores, _info.num_subcores
_NW = _NC * _NS
_B_PER_W = BATCH // _NW


def _make_sc_gather():
  mesh = plsc.VectorSubcoreMesh(core_axis_name="c", subcore_axis_name="s")

  @functools.partial(
      pl.kernel,
      mesh=mesh,
      compiler_params=pltpu.CompilerParams(use_tc_tiling_on_sc=True),
      out_type=jax.ShapeDtypeStruct((BATCH, EMBED_PAD), jnp.float32),
      scratch_types=[
          pltpu.VMEM((_B_PER_W,), jnp.int32),
          pltpu.VMEM((_B_PER_W, EMBED_PAD), jnp.float32),
          pltpu.SemaphoreType.DMA,
      ],
  )
  def gather_kernel(table_hbm, idx_hbm, out_hbm, idx_v, rows_v, sem):
    wid = lax.axis_index("s") * _NC + lax.axis_index("c")
    base = wid * _B_PER_W
    pltpu.sync_copy(idx_hbm.at[pl.ds(base, _B_PER_W)], idx_v)
    pltpu.async_copy(table_hbm.at[idx_v], rows_v, sem).wait()
    pltpu.sync_copy(rows_v, out_hbm.at[pl.ds(base, _B_PER_W)])

  return gather_kernel


_sc_gather = _make_sc_gather()


_VT = 4096


def _proj_body(h4_ref, w_ref, b_ref, out_ref, ht_scr):
  @pl.when(pl.program_id(0) == 0)
  def _():
    ht_scr[...] = h4_ref[:, :EMBED].T

  acc = lax.dot_general(
      w_ref[...],
      ht_scr[...],
      dimension_numbers=(((0,), (0,)), ((), ())),
      preferred_element_type=jnp.float32,
  )
  bias = b_ref[...].reshape(1, _VT).T
  out_ref[...] = acc + bias


def _projection(h4, W, b):
  grid = (pl.cdiv(VOCAB, _VT),)
  out_t = pl.pallas_call(
      _proj_body,
      grid=grid,
      in_specs=[
          pl.BlockSpec((BATCH, EMBED_PAD), lambda j: (0, 0)),
          pl.BlockSpec((EMBED, _VT), lambda j: (0, j)),
          pl.BlockSpec((_VT,), lambda j: (j,)),
      ],
      out_specs=pl.BlockSpec((_VT, BATCH), lambda j: (j, 0)),
      out_shape=jax.ShapeDtypeStruct((VOCAB, BATCH), jnp.float32),
      scratch_shapes=[pltpu.VMEM((EMBED, BATCH), jnp.float32)],
  )(h4, W, b)
  return out_t.T


def kernel(x, emb_table, W, b):
  emb_pad = jnp.pad(emb_table, ((0, 0), (0, EMBED_PAD - EMBED)))
  h4 = _sc_gather(emb_pad, x.astype(jnp.int32))
  return _projection(h4, W, b)


# speedup vs baseline: 3.1734x; 1.1336x over previous
import functools

import jax
import jax.numpy as jnp
from jax import lax
from jax.experimental import pallas as pl
from jax.experimental.pallas import tpu as pltpu
from jax.experimental.pallas import tpu_sc as plsc

VOCAB = 100000
EMBED = 32
SLAB = 8
NSLAB = VOCAB // SLAB
BATCH = 1024


_info = plsc.get_sparse_core_info()
_NC, _NS = _info.num_cores, _info.num_subcores
_NW = _NC * _NS
_B_PER_W = BATCH // _NW


def _make_sc_gather():
  mesh = plsc.VectorSubcoreMesh(core_axis_name="c", subcore_axis_name="s")

  @functools.partial(
      pl.kernel,
      mesh=mesh,
      compiler_params=pltpu.CompilerParams(use_tc_tiling_on_sc=True),
      out_type=jax.ShapeDtypeStruct((BATCH, SLAB, EMBED), jnp.float32),
      scratch_types=[
          pltpu.VMEM((_B_PER_W,), jnp.float32),
          pltpu.VMEM((_B_PER_W, SLAB, EMBED), jnp.float32),
          pltpu.SemaphoreType.DMA,
      ],
  )
  def gather_kernel(table_hbm, idx_hbm, out_hbm, idx_v, slabs_v, sem):
    wid = lax.axis_index("s") * _NC + lax.axis_index("c")
    base = wid * _B_PER_W
    pltpu.sync_copy(idx_hbm.at[pl.ds(base, _B_PER_W)], idx_v)
    copies = []
    for k in range(_B_PER_W // 16):
      iv = lax.bitcast_convert_type(idx_v[pl.ds(16 * k, 16)], jnp.int32)
      tv = lax.shift_right_logical(iv, 3)
      for j in range(16):
        i = 16 * k + j
        copies.append(
            pltpu.async_copy(
                table_hbm.at[pl.ds(tv[j], 1)],
                slabs_v.at[pl.ds(i, 1)],
                sem,
            )
        )
    for c in copies:
      c.wait()
    pltpu.sync_copy(slabs_v, out_hbm.at[pl.ds(base, _B_PER_W)])

  return gather_kernel


_sc_gather = _make_sc_gather()


_VT = 2048


def _proj_body(x_ref, h8_ref, w_ref, b_ref, out_ref, ht_scr):
  @pl.when(pl.program_id(0) == 0)
  def _():
    sub = lax.rem(x_ref[...], jnp.int32(SLAB))
    onehot = (
        sub[:, None, None] == lax.broadcasted_iota(jnp.int32, (1, SLAB, 1), 1)
    ).astype(jnp.float32)
    h = jnp.sum(h8_ref[...] * onehot, axis=1)
    ht_scr[...] = h.T

  acc = lax.dot_general(
      w_ref[...],
      ht_scr[...],
      dimension_numbers=(((0,), (0,)), ((), ())),
      preferred_element_type=jnp.float32,
  )
  bias = b_ref[...].reshape(1, _VT).T
  out_ref[...] = acc + bias


def _projection(x, h8, W, b):
  grid = (pl.cdiv(VOCAB, _VT),)
  out_t = pl.pallas_call(
      _proj_body,
      grid=grid,
      in_specs=[
          pl.BlockSpec((BATCH,), lambda j: (0,)),
          pl.BlockSpec((BATCH, SLAB, EMBED), lambda j: (0, 0, 0)),
          pl.BlockSpec((EMBED, _VT), lambda j: (0, j)),
          pl.BlockSpec((_VT,), lambda j: (j,)),
      ],
      out_specs=pl.BlockSpec((_VT, BATCH), lambda j: (j, 0)),
      out_shape=jax.ShapeDtypeStruct((VOCAB, BATCH), jnp.float32),
      scratch_shapes=[pltpu.VMEM((EMBED, BATCH), jnp.float32)],
  )(x, h8, W, b)
  return out_t.T


def kernel(x, emb_table, W, b):
  xi = x.astype(jnp.int32)
  xf = lax.bitcast_convert_type(xi, jnp.float32)
  table_slabs = emb_table.reshape(NSLAB, SLAB, EMBED)
  h8 = _sc_gather(table_slabs, xf)
  return _projection(xi, h8, W, b)
